# baseline (speedup 1.0000x reference)
import functools

import jax
import jax.numpy as jnp
from jax import lax
from jax.experimental import pallas as pl
from jax.experimental.pallas import tpu as pltpu
from jax.experimental.pallas import tpu_sc as plsc

_D = 768
_B = 16384
_NC = 2
_NS = 16
_NW = _NC * _NS
_BPW = _B // _NW
_C = 128
_NBUF = 1
_NCHUNK = _BPW // _C


def _embed_body(ids_hbm, table_hbm, out_hbm, idx_v, bufs, *sems):
    gsems = sems[:_NBUF]
    wsems = sems[_NBUF:]
    wid = lax.axis_index("s") * _NC + lax.axis_index("c")
    base = wid * _BPW

    pltpu.sync_copy(ids_hbm.at[pl.ds(base, _BPW)], idx_v)

    ghandles = [None] * _NCHUNK
    whandles = [None] * _NCHUNK

    def start_gather(g):
        b = g % _NBUF
        ghandles[g] = pltpu.async_copy(
            table_hbm.at[idx_v.at[pl.ds(g * _C, _C)]], bufs.at[b], gsems[b])

    def start_write(g):
        b = g % _NBUF
        whandles[g] = pltpu.async_copy(
            bufs.at[b], out_hbm.at[pl.ds(base + g * _C, _C)], wsems[b])

    if _NBUF == 1:
        for g in range(_NCHUNK):
            start_gather(g)
            ghandles[g].wait()
            start_write(g)
            whandles[g].wait()
        return

    for g in range(min(_NBUF - 1, _NCHUNK)):
        start_gather(g)

    waited_writes = set()
    for g in range(_NCHUNK):
        ghandles[g].wait()
        start_write(g)
        h = g + _NBUF - 1
        if h < _NCHUNK:
            if g >= 1:
                whandles[g - 1].wait()
                waited_writes.add(g - 1)
            start_gather(h)

    for g in range(_NCHUNK):
        if g not in waited_writes:
            whandles[g].wait()


@jax.jit
def _embed(ids, table):
    mesh = plsc.VectorSubcoreMesh(core_axis_name="c", subcore_axis_name="s")
    f = functools.partial(
        pl.kernel,
        mesh=mesh,
        out_type=jax.ShapeDtypeStruct((_B, _D), jnp.float32),
        scratch_types=[
            pltpu.VMEM((_BPW,), jnp.int32),
            pltpu.VMEM((_NBUF, _C, _D), jnp.float32),
        ] + [pltpu.SemaphoreType.DMA] * (2 * _NBUF),
    )(_embed_body)
    return f(ids, table)


def kernel(input_ids, table):
    ids = input_ids.astype(jnp.int32)
    return _embed(ids, table)


# speedup vs baseline: 1.0242x; 1.0242x over previous
import functools

import jax
import jax.numpy as jnp
from jax import lax
from jax.experimental import pallas as pl
from jax.experimental.pallas import tpu as pltpu
from jax.experimental.pallas import tpu_sc as plsc

_D = 768
_B = 16384
_NC = 2
_NS = 16
_NW = _NC * _NS
_BPW = _B // _NW
_C = 32
_NBUF = 3
_NCHUNK = _BPW // _C


def _embed_body(ids_hbm, table_hbm, out_hbm, idx_v, bufs, sp, *sems):
    gsems = sems[:_NBUF]
    ssems = sems[_NBUF:_NBUF + 2]
    wsems = sems[_NBUF + 2:]
    sid = lax.axis_index("s")
    wid = sid * _NC + lax.axis_index("c")
    base = wid * _BPW
    pltpu.sync_copy(ids_hbm.at[pl.ds(base, _BPW)], idx_v)
    ghandles = [None] * _NCHUNK
    shandles = [None] * _NCHUNK
    whandles = [None] * _NCHUNK
    def start_gather(g):
        b = g % _NBUF
        ghandles[g] = pltpu.async_copy(
            table_hbm.at[idx_v.at[pl.ds(g * _C, _C)]], bufs.at[b], gsems[b])
    for g in range(2):
        start_gather(g)
    for g in range(_NCHUNK):
        b = g % _NBUF
        q = g % 2
        ghandles[g].wait()
        if g >= 2:
            whandles[g - 2].wait()
        shandles[g] = pltpu.async_copy(bufs.at[b], sp.at[sid, q], ssems[q])
        shandles[g].wait()
        whandles[g] = pltpu.async_copy(
            sp.at[sid, q], out_hbm.at[pl.ds(base + g * _C, _C)], wsems[q])
        h = g + 2
        if h < _NCHUNK:
            start_gather(h)
    whandles[_NCHUNK - 2].wait()
    whandles[_NCHUNK - 1].wait()


@jax.jit
def _embed(ids, table):
    mesh = plsc.VectorSubcoreMesh(core_axis_name="c", subcore_axis_name="s")
    f = functools.partial(
        pl.kernel,
        mesh=mesh,
        out_type=jax.ShapeDtypeStruct((_B, _D), jnp.float32),
        scratch_types=[
            pltpu.VMEM((_BPW,), jnp.int32),
            pltpu.VMEM((_NBUF, _C, _D), jnp.float32),
            pltpu.VMEM_SHARED((_NS, 2, _C, _D), jnp.float32),
        ] + [pltpu.SemaphoreType.DMA] * (_NBUF + 4),
    )(_embed_body)
    return f(ids, table)


def kernel(input_ids, table):
    ids = input_ids.astype(jnp.int32)
    return _embed(ids, table)
